# baseline (speedup 1.0000x reference)
import jax
import jax.numpy as jnp
from jax import lax
from jax.experimental import pallas as pl
from jax.experimental.pallas import tpu as pltpu
from jax.experimental.pallas import tpu_sc as plsc

N = 10000
E = 320000
EP = 327680
D = 128
NQ = 4
DH = D // NQ
QPC = NQ // 2

NC = 2
NS = 16
EPT = EP // NS

C = 512
SUB = 128
NSUB = C // SUB
CHUNKS = EPT // C
IDX_ROWS = EPT // SUB
ROWS_PER_TILE = N // NS
ZROWS = 125
DHP = DH + 1


def _sc_body(h_hbm, src_hbm, dst_hbm, vals_hbm, out_hbm,
             src_v, dst_v, vals_v, rows_a, rows_b, zbuf_v,
             gsem_a, gsem_b, ssem_a, ssem_b, acc):
    cid = lax.axis_index("c")
    sid = lax.axis_index("s")
    row_lo = sid * ROWS_PER_TILE
    lane = lax.iota(jnp.int32, 16)

    pltpu.sync_copy(src_hbm.at[pl.ds(sid * IDX_ROWS, IDX_ROWS)], src_v)
    pltpu.sync_copy(dst_hbm.at[pl.ds(sid * IDX_ROWS, IDX_ROWS)], dst_v)
    pltpu.sync_copy(vals_hbm.at[pl.ds(sid * EPT, EPT)], vals_v)

    zero16 = jnp.zeros((16,), jnp.float32)
    for r in range(ZROWS):
        for k in range(DH // 16):
            zbuf_v[r, pl.ds(k * 16, 16)] = zero16

    def scale(buf, c):
        def scale_body(g, _):
            vals_vec = vals_v[pl.ds(c * C + g * 16, 16)]
            for j in range(16):
                e = g * 16 + j
                val = vals_vec[jnp.full((16,), j, jnp.int32)]
                for k in range(DH // 16):
                    buf[e, pl.ds(k * 16, 16)] = buf[e, pl.ds(k * 16, 16)] * val
            return 0
        lax.fori_loop(0, C // 16, scale_body, 0)

    for q in range(QPC):
        for m in range(ROWS_PER_TILE // ZROWS):
            pltpu.sync_copy(zbuf_v, acc.at[pl.ds(row_lo + m * ZROWS, ZROWS)])
        plsc.subcore_barrier()

        hq = h_hbm.at[cid * QPC + q]

        def fire_gather(c, buf, sem):
            for j in range(NSUB):
                pltpu.make_async_copy(
                    hq.at[src_v.at[c * NSUB + j]],
                    buf.at[pl.ds(j * SUB, SUB)], sem).start()

        def wait_gather(c, buf, sem):
            for j in range(NSUB):
                pltpu.make_async_copy(
                    hq.at[src_v.at[c * NSUB + j]],
                    buf.at[pl.ds(j * SUB, SUB)], sem).wait()

        def fire_scatter(c, buf, sem):
            for j in range(NSUB):
                pltpu.make_async_copy(
                    buf.at[pl.ds(j * SUB, SUB)],
                    acc.at[dst_v.at[c * NSUB + j]], sem).start(add=True)

        def wait_scatter(c, buf, sem):
            for j in range(NSUB):
                pltpu.make_async_copy(
                    buf.at[pl.ds(j * SUB, SUB)],
                    acc.at[dst_v.at[c * NSUB + j]], sem).wait()

        fire_gather(0, rows_a, gsem_a)

        def pair_body(t, _):
            c0 = 2 * t
            c1 = c0 + 1
            wait_gather(c0, rows_a, gsem_a)

            @pl.when(t > 0)
            def _():
                wait_scatter(c1 - 2, rows_b, ssem_b)
            fire_gather(c1, rows_b, gsem_b)
            scale(rows_a, c0)
            fire_scatter(c0, rows_a, ssem_a)
            wait_gather(c1, rows_b, gsem_b)

            @pl.when(t + 1 < CHUNKS // 2)
            def _():
                wait_scatter(c0, rows_a, ssem_a)
                fire_gather(c0 + 2, rows_a, gsem_a)
            scale(rows_b, c1)
            fire_scatter(c1, rows_b, ssem_b)
            return 0

        lax.fori_loop(0, CHUNKS // 2, pair_body, 0)
        wait_scatter(CHUNKS - 2, rows_a, ssem_a)
        wait_scatter(CHUNKS - 1, rows_b, ssem_b)
        plsc.subcore_barrier()

        col_lo = (cid * QPC + q) * DH
        for m in range(ROWS_PER_TILE // ZROWS):
            buf = rows_a if m % 2 == 0 else rows_b
            r_lo = row_lo + m * ZROWS
            pltpu.sync_copy(acc.at[pl.ds(r_lo, ZROWS)],
                            buf.at[pl.ds(0, ZROWS)])

            def leaky_body(r, _):
                for k in range(DH // 16):
                    v = buf[r, pl.ds(k * 16, 16)]
                    buf[r, pl.ds(k * 16, 16)] = jnp.where(v >= 0, v,
                                                          0.01 * v)
                return 0
            lax.fori_loop(0, ZROWS, leaky_body, 0,
                          unroll=4)
            pltpu.make_async_copy(
                buf.at[pl.ds(0, ZROWS)],
                out_hbm.at[pl.ds(r_lo, ZROWS), pl.ds(col_lo, DH)],
                gsem_a if m % 2 == 0 else gsem_b).start()
            if m >= 2:
                p_lo = row_lo + (m - 2) * ZROWS
                pltpu.make_async_copy(
                    buf.at[pl.ds(0, ZROWS)],
                    out_hbm.at[pl.ds(p_lo, ZROWS), pl.ds(col_lo, DH)],
                    gsem_a if m % 2 == 0 else gsem_b).wait()
        for m in (3, 4):
            buf = rows_a if m % 2 == 0 else rows_b
            r_lo = row_lo + m * ZROWS
            pltpu.make_async_copy(
                buf.at[pl.ds(0, ZROWS)],
                out_hbm.at[pl.ds(r_lo, ZROWS), pl.ds(col_lo, DH)],
                gsem_a if m % 2 == 0 else gsem_b).wait()
        if q + 1 < QPC:
            plsc.subcore_barrier()


def _sc_aggregate(h, src2d, dst2d, vals):
    mesh = plsc.VectorSubcoreMesh(core_axis_name="c", subcore_axis_name="s")
    return pl.kernel(
        _sc_body,
        out_type=jax.ShapeDtypeStruct((N, D), jnp.float32),
        mesh=mesh,
        compiler_params=pltpu.CompilerParams(use_tc_tiling_on_sc=False,
                                             needs_layout_passes=False),
        scratch_types=[
            pltpu.VMEM((IDX_ROWS, SUB), jnp.int32),
            pltpu.VMEM((IDX_ROWS, SUB), jnp.int32),
            pltpu.VMEM((EPT,), jnp.float32),
            pltpu.VMEM((C, DH), jnp.float32),
            pltpu.VMEM((C, DH), jnp.float32),
            pltpu.VMEM((ZROWS, DH), jnp.float32),
            pltpu.SemaphoreType.DMA,
            pltpu.SemaphoreType.DMA,
            pltpu.SemaphoreType.DMA,
            pltpu.SemaphoreType.DMA,
            pltpu.VMEM_SHARED((N, DH), jnp.float32),
        ],
    )(h, src2d, dst2d, vals)


def _mm_body(x_ref, w_ref, o_ref):
    h = jnp.dot(x_ref[...], w_ref[...], preferred_element_type=jnp.float32)
    for q in range(NQ):
        o_ref[q] = h[:, q * DH:(q + 1) * DH]


def _matmul(x, W):
    blk = 2000
    return pl.pallas_call(
        _mm_body,
        grid=(N // blk,),
        in_specs=[
            pl.BlockSpec((blk, D), lambda i: (i, 0)),
            pl.BlockSpec((D, D), lambda i: (0, 0)),
        ],
        out_specs=pl.BlockSpec((NQ, blk, DH), lambda i: (0, i, 0)),
        out_shape=jax.ShapeDtypeStruct((NQ, N, DH), jnp.float32),
    )(x, W)


def kernel(x, edge_index, adj_vals, W):
    h = _matmul(x, W)
    pad_i = jnp.zeros((EP - E,), jnp.int32)
    src2d = jnp.concatenate([edge_index[0], pad_i]).reshape(EP // SUB, SUB)
    dst2d = jnp.concatenate([edge_index[1], pad_i]).reshape(EP // SUB, SUB)
    vals_p = jnp.concatenate([adj_vals, jnp.zeros((EP - E,), jnp.float32)])
    return _sc_aggregate(h, src2d, dst2d, vals_p)


# speedup vs baseline: 1.7747x; 1.7747x over previous
import jax
import jax.numpy as jnp
from jax import lax
from jax.experimental import pallas as pl
from jax.experimental.pallas import tpu as pltpu
from jax.experimental.pallas import tpu_sc as plsc

N = 10000
E = 320000
D = 128
NQ = 4
DH = D // NQ
QPC = NQ // 2

NC = 2
NS = 16
EPT = E // NS

C = 400
SUB = 100
NSUB = C // SUB
CHUNKS = EPT // C
IDX_ROWS = EPT // SUB
ROWS_PER_TILE = N // NS
ZROWS = 125
DHP = DH + 1


def _sc_body(h_hbm, src_hbm, dst_hbm, vals_hbm, out_hbm,
             src_v, dst_v, vals_v, rows_a, rows_b, zbuf_v,
             gsem_a, gsem_b, ssem_a, ssem_b, acc):
    cid = lax.axis_index("c")
    sid = lax.axis_index("s")
    row_lo = sid * ROWS_PER_TILE
    lane = lax.iota(jnp.int32, 16)

    pltpu.sync_copy(src_hbm.at[pl.ds(sid * IDX_ROWS, IDX_ROWS)], src_v)
    pltpu.sync_copy(dst_hbm.at[pl.ds(sid * IDX_ROWS, IDX_ROWS)], dst_v)
    pltpu.sync_copy(vals_hbm.at[pl.ds(sid * EPT, EPT)], vals_v)

    zero16 = jnp.zeros((16,), jnp.float32)
    for r in range(ZROWS):
        for k in range(DH // 16):
            zbuf_v[r, pl.ds(k * 16, 16)] = zero16

    def scale(buf, c):
        def scale_body(g, _):
            vals_vec = vals_v[pl.ds(c * C + g * 16, 16)]
            for j in range(16):
                e = g * 16 + j
                val = vals_vec[jnp.full((16,), j, jnp.int32)]
                for k in range(DH // 16):
                    buf[e, pl.ds(k * 16, 16)] = buf[e, pl.ds(k * 16, 16)] * val
            return 0
        lax.fori_loop(0, C // 16, scale_body, 0)

    for q in range(QPC):
        for m in range(ROWS_PER_TILE // ZROWS):
            pltpu.sync_copy(zbuf_v, acc.at[pl.ds(row_lo + m * ZROWS, ZROWS)])
        plsc.subcore_barrier()

        hq = h_hbm.at[cid * QPC + q]

        def fire_gather(c, buf, sem):
            for j in range(NSUB):
                pltpu.make_async_copy(
                    hq.at[src_v.at[c * NSUB + j]],
                    buf.at[pl.ds(j * SUB, SUB)], sem).start()

        def wait_gather(c, buf, sem):
            for j in range(NSUB):
                pltpu.make_async_copy(
                    hq.at[src_v.at[c * NSUB + j]],
                    buf.at[pl.ds(j * SUB, SUB)], sem).wait()

        def fire_scatter(c, buf, sem):
            for j in range(NSUB):
                pltpu.make_async_copy(
                    buf.at[pl.ds(j * SUB, SUB)],
                    acc.at[dst_v.at[c * NSUB + j]], sem).start(add=True)

        def wait_scatter(c, buf, sem):
            for j in range(NSUB):
                pltpu.make_async_copy(
                    buf.at[pl.ds(j * SUB, SUB)],
                    acc.at[dst_v.at[c * NSUB + j]], sem).wait()

        fire_gather(0, rows_a, gsem_a)

        def pair_body(t, _):
            c0 = 2 * t
            c1 = c0 + 1
            wait_gather(c0, rows_a, gsem_a)

            @pl.when(t > 0)
            def _():
                wait_scatter(c1 - 2, rows_b, ssem_b)
            fire_gather(c1, rows_b, gsem_b)
            scale(rows_a, c0)
            fire_scatter(c0, rows_a, ssem_a)
            wait_gather(c1, rows_b, gsem_b)

            @pl.when(t + 1 < CHUNKS // 2)
            def _():
                wait_scatter(c0, rows_a, ssem_a)
                fire_gather(c0 + 2, rows_a, gsem_a)
            scale(rows_b, c1)
            fire_scatter(c1, rows_b, ssem_b)
            return 0

        lax.fori_loop(0, CHUNKS // 2, pair_body, 0)
        wait_scatter(CHUNKS - 2, rows_a, ssem_a)
        wait_scatter(CHUNKS - 1, rows_b, ssem_b)
        plsc.subcore_barrier()

        col_lo = (cid * QPC + q) * DH
        for m in range(ROWS_PER_TILE // ZROWS):
            buf = rows_a if m % 2 == 0 else rows_b
            r_lo = row_lo + m * ZROWS
            pltpu.sync_copy(acc.at[pl.ds(r_lo, ZROWS)],
                            buf.at[pl.ds(0, ZROWS)])

            def leaky_body(r, _):
                for k in range(DH // 16):
                    v = buf[r, pl.ds(k * 16, 16)]
                    buf[r, pl.ds(k * 16, 16)] = jnp.where(v >= 0, v,
                                                          0.01 * v)
                return 0
            lax.fori_loop(0, ZROWS, leaky_body, 0,
                          unroll=4)
            pltpu.make_async_copy(
                buf.at[pl.ds(0, ZROWS)],
                out_hbm.at[pl.ds(r_lo, ZROWS), pl.ds(col_lo, DH)],
                gsem_a if m % 2 == 0 else gsem_b).start()
            if m >= 2:
                p_lo = row_lo + (m - 2) * ZROWS
                pltpu.make_async_copy(
                    buf.at[pl.ds(0, ZROWS)],
                    out_hbm.at[pl.ds(p_lo, ZROWS), pl.ds(col_lo, DH)],
                    gsem_a if m % 2 == 0 else gsem_b).wait()
        for m in (3, 4):
            buf = rows_a if m % 2 == 0 else rows_b
            r_lo = row_lo + m * ZROWS
            pltpu.make_async_copy(
                buf.at[pl.ds(0, ZROWS)],
                out_hbm.at[pl.ds(r_lo, ZROWS), pl.ds(col_lo, DH)],
                gsem_a if m % 2 == 0 else gsem_b).wait()
        if q + 1 < QPC:
            plsc.subcore_barrier()


def _sc_aggregate(h, src2d, dst2d, vals):
    mesh = plsc.VectorSubcoreMesh(core_axis_name="c", subcore_axis_name="s")
    return pl.kernel(
        _sc_body,
        out_type=jax.ShapeDtypeStruct((N, D), jnp.float32),
        mesh=mesh,
        compiler_params=pltpu.CompilerParams(use_tc_tiling_on_sc=False,
                                             needs_layout_passes=False),
        scratch_types=[
            pltpu.VMEM((IDX_ROWS, SUB), jnp.int32),
            pltpu.VMEM((IDX_ROWS, SUB), jnp.int32),
            pltpu.VMEM((EPT,), jnp.float32),
            pltpu.VMEM((C, DH), jnp.float32),
            pltpu.VMEM((C, DH), jnp.float32),
            pltpu.VMEM((ZROWS, DH), jnp.float32),
            pltpu.SemaphoreType.DMA,
            pltpu.SemaphoreType.DMA,
            pltpu.SemaphoreType.DMA,
            pltpu.SemaphoreType.DMA,
            pltpu.VMEM_SHARED((N, DH), jnp.float32),
        ],
    )(h, src2d, dst2d, vals)


def _mm_body(x_ref, w_ref, o_ref):
    h = jnp.dot(x_ref[...], w_ref[...], preferred_element_type=jnp.float32)
    for q in range(NQ):
        o_ref[q] = h[:, q * DH:(q + 1) * DH]


def _matmul(x, W):
    blk = 2000
    return pl.pallas_call(
        _mm_body,
        grid=(N // blk,),
        in_specs=[
            pl.BlockSpec((blk, D), lambda i: (i, 0)),
            pl.BlockSpec((D, D), lambda i: (0, 0)),
        ],
        out_specs=pl.BlockSpec((NQ, blk, DH), lambda i: (0, i, 0)),
        out_shape=jax.ShapeDtypeStruct((NQ, N, DH), jnp.float32),
    )(x, W)


def kernel(x, edge_index, adj_vals, W):
    h = _matmul(x, W)
    src2d = edge_index[0].reshape(E // SUB, SUB)
    dst2d = edge_index[1].reshape(E // SUB, SUB)
    return _sc_aggregate(h, src2d, dst2d, adj_vals)


# speedup vs baseline: 1.7992x; 1.0138x over previous
import jax
import jax.numpy as jnp
from jax import lax
from jax.experimental import pallas as pl
from jax.experimental.pallas import tpu as pltpu
from jax.experimental.pallas import tpu_sc as plsc

N = 10000
E = 320000
D = 128
NQ = 4
DH = D // NQ
QPC = NQ // 2

NC = 2
NS = 16
EPT = E // NS

C = 400
SUB = 100
NSUB = C // SUB
CHUNKS = EPT // C
IDX_ROWS = EPT // SUB
ROWS_PER_TILE = N // NS
ZROWS = 125
DHP = DH + 1


def _sc_body(h_hbm, src_hbm, dst_hbm, vals_hbm, out_hbm,
             src_v, dst_v, vals_v, rows_a, rows_b, zbuf_v,
             gsem_a, gsem_b, ssem_a, ssem_b, acc):
    cid = lax.axis_index("c")
    sid = lax.axis_index("s")
    row_lo = sid * ROWS_PER_TILE
    lane = lax.iota(jnp.int32, 16)

    pltpu.sync_copy(src_hbm.at[pl.ds(sid * IDX_ROWS, IDX_ROWS)], src_v)
    pltpu.sync_copy(dst_hbm.at[pl.ds(sid * IDX_ROWS, IDX_ROWS)], dst_v)
    pltpu.sync_copy(vals_hbm.at[pl.ds(sid * EPT, EPT)], vals_v)

    zero16 = jnp.zeros((16,), jnp.float32)
    for r in range(ZROWS):
        for k in range(DH // 16):
            zbuf_v[r, pl.ds(k * 16, 16)] = zero16

    def scale(buf, c):
        @plsc.parallel_loop(0, C // 16, unroll=2)
        def scale_body(g):
            vals_vec = vals_v[pl.ds(c * C + g * 16, 16)]
            for j in range(16):
                e = g * 16 + j
                val = vals_vec[jnp.full((16,), j, jnp.int32)]
                for k in range(DH // 16):
                    buf[e, pl.ds(k * 16, 16)] = buf[e, pl.ds(k * 16, 16)] * val

    for q in range(QPC):
        for m in range(ROWS_PER_TILE // ZROWS):
            pltpu.sync_copy(zbuf_v, acc.at[pl.ds(row_lo + m * ZROWS, ZROWS)])
        plsc.subcore_barrier()

        hq = h_hbm.at[cid * QPC + q]

        def fire_gather(c, buf, sem):
            for j in range(NSUB):
                pltpu.make_async_copy(
                    hq.at[src_v.at[c * NSUB + j]],
                    buf.at[pl.ds(j * SUB, SUB)], sem).start()

        def wait_gather(c, buf, sem):
            for j in range(NSUB):
                pltpu.make_async_copy(
                    hq.at[src_v.at[c * NSUB + j]],
                    buf.at[pl.ds(j * SUB, SUB)], sem).wait()

        def fire_scatter(c, buf, sem):
            for j in range(NSUB):
                pltpu.make_async_copy(
                    buf.at[pl.ds(j * SUB, SUB)],
                    acc.at[dst_v.at[c * NSUB + j]], sem).start(add=True)

        def wait_scatter(c, buf, sem):
            for j in range(NSUB):
                pltpu.make_async_copy(
                    buf.at[pl.ds(j * SUB, SUB)],
                    acc.at[dst_v.at[c * NSUB + j]], sem).wait()

        fire_gather(0, rows_a, gsem_a)

        def pair_body(t, _):
            c0 = 2 * t
            c1 = c0 + 1
            wait_gather(c0, rows_a, gsem_a)

            @pl.when(t > 0)
            def _():
                wait_scatter(c1 - 2, rows_b, ssem_b)
            fire_gather(c1, rows_b, gsem_b)
            scale(rows_a, c0)
            fire_scatter(c0, rows_a, ssem_a)
            wait_gather(c1, rows_b, gsem_b)

            @pl.when(t + 1 < CHUNKS // 2)
            def _():
                wait_scatter(c0, rows_a, ssem_a)
                fire_gather(c0 + 2, rows_a, gsem_a)
            scale(rows_b, c1)
            fire_scatter(c1, rows_b, ssem_b)
            return 0

        lax.fori_loop(0, CHUNKS // 2, pair_body, 0)
        wait_scatter(CHUNKS - 2, rows_a, ssem_a)
        wait_scatter(CHUNKS - 1, rows_b, ssem_b)
        plsc.subcore_barrier()

        col_lo = (cid * QPC + q) * DH
        for m in range(ROWS_PER_TILE // ZROWS):
            buf = rows_a if m % 2 == 0 else rows_b
            r_lo = row_lo + m * ZROWS
            pltpu.sync_copy(acc.at[pl.ds(r_lo, ZROWS)],
                            buf.at[pl.ds(0, ZROWS)])

            @plsc.parallel_loop(0, ZROWS, unroll=4)
            def leaky_body(r):
                for k in range(DH // 16):
                    v = buf[r, pl.ds(k * 16, 16)]
                    buf[r, pl.ds(k * 16, 16)] = jnp.where(v >= 0, v,
                                                          0.01 * v)
            pltpu.make_async_copy(
                buf.at[pl.ds(0, ZROWS)],
                out_hbm.at[pl.ds(r_lo, ZROWS), pl.ds(col_lo, DH)],
                gsem_a if m % 2 == 0 else gsem_b).start()
            if m >= 2:
                p_lo = row_lo + (m - 2) * ZROWS
                pltpu.make_async_copy(
                    buf.at[pl.ds(0, ZROWS)],
                    out_hbm.at[pl.ds(p_lo, ZROWS), pl.ds(col_lo, DH)],
                    gsem_a if m % 2 == 0 else gsem_b).wait()
        for m in (3, 4):
            buf = rows_a if m % 2 == 0 else rows_b
            r_lo = row_lo + m * ZROWS
            pltpu.make_async_copy(
                buf.at[pl.ds(0, ZROWS)],
                out_hbm.at[pl.ds(r_lo, ZROWS), pl.ds(col_lo, DH)],
                gsem_a if m % 2 == 0 else gsem_b).wait()
        if q + 1 < QPC:
            plsc.subcore_barrier()


def _sc_aggregate(h, src2d, dst2d, vals):
    mesh = plsc.VectorSubcoreMesh(core_axis_name="c", subcore_axis_name="s")
    return pl.kernel(
        _sc_body,
        out_type=jax.ShapeDtypeStruct((N, D), jnp.float32),
        mesh=mesh,
        compiler_params=pltpu.CompilerParams(use_tc_tiling_on_sc=False,
                                             needs_layout_passes=False),
        scratch_types=[
            pltpu.VMEM((IDX_ROWS, SUB), jnp.int32),
            pltpu.VMEM((IDX_ROWS, SUB), jnp.int32),
            pltpu.VMEM((EPT,), jnp.float32),
            pltpu.VMEM((C, DH), jnp.float32),
            pltpu.VMEM((C, DH), jnp.float32),
            pltpu.VMEM((ZROWS, DH), jnp.float32),
            pltpu.SemaphoreType.DMA,
            pltpu.SemaphoreType.DMA,
            pltpu.SemaphoreType.DMA,
            pltpu.SemaphoreType.DMA,
            pltpu.VMEM_SHARED((N, DH), jnp.float32),
        ],
    )(h, src2d, dst2d, vals)


def _mm_body(x_ref, w_ref, o_ref):
    h = jnp.dot(x_ref[...], w_ref[...], preferred_element_type=jnp.float32)
    for q in range(NQ):
        o_ref[q] = h[:, q * DH:(q + 1) * DH]


def _matmul(x, W):
    blk = 2000
    return pl.pallas_call(
        _mm_body,
        grid=(N // blk,),
        in_specs=[
            pl.BlockSpec((blk, D), lambda i: (i, 0)),
            pl.BlockSpec((D, D), lambda i: (0, 0)),
        ],
        out_specs=pl.BlockSpec((NQ, blk, DH), lambda i: (0, i, 0)),
        out_shape=jax.ShapeDtypeStruct((NQ, N, DH), jnp.float32),
    )(x, W)


def kernel(x, edge_index, adj_vals, W):
    h = _matmul(x, W)
    src2d = edge_index[0].reshape(E // SUB, SUB)
    dst2d = edge_index[1].reshape(E // SUB, SUB)
    return _sc_aggregate(h, src2d, dst2d, adj_vals)


# speedup vs baseline: 1.8801x; 1.0450x over previous
import jax
import jax.numpy as jnp
from jax import lax
from jax.experimental import pallas as pl
from jax.experimental.pallas import tpu as pltpu
from jax.experimental.pallas import tpu_sc as plsc

N = 10000
E = 320000
D = 128
NQ = 4
DH = D // NQ
QPC = NQ // 2

NC = 2
NS = 16
EPT = E // NS

C = 400
SUB = 100
NSUB = C // SUB
CHUNKS = EPT // C
IDX_ROWS = EPT // SUB
ROWS_PER_TILE = N // NS
ZROWS = 125
DHP = DH + 1


def _sc_body(h_hbm, edge_hbm, vals_hbm, out_hbm,
             src_v, dst_v, vals_v, rows_a, rows_b, zbuf_v,
             gsem_a, gsem_b, ssem_a, ssem_b, acc):
    cid = lax.axis_index("c")
    sid = lax.axis_index("s")
    row_lo = sid * ROWS_PER_TILE
    lane = lax.iota(jnp.int32, 16)

    pltpu.sync_copy(edge_hbm.at[0, pl.ds(sid * IDX_ROWS, IDX_ROWS)], src_v)
    pltpu.sync_copy(edge_hbm.at[1, pl.ds(sid * IDX_ROWS, IDX_ROWS)], dst_v)
    pltpu.sync_copy(vals_hbm.at[pl.ds(sid * EPT, EPT)], vals_v)

    zero16 = jnp.zeros((16,), jnp.float32)
    for r in range(ZROWS):
        for k in range(DH // 16):
            zbuf_v[r, pl.ds(k * 16, 16)] = zero16

    def scale(buf, c):
        @plsc.parallel_loop(0, C // 16, unroll=2)
        def scale_body(g):
            vals_vec = vals_v[pl.ds(c * C + g * 16, 16)]
            for j in range(16):
                e = g * 16 + j
                val = vals_vec[jnp.full((16,), j, jnp.int32)]
                for k in range(DH // 16):
                    buf[e, pl.ds(k * 16, 16)] = buf[e, pl.ds(k * 16, 16)] * val

    for q in range(QPC):
        for m in range(ROWS_PER_TILE // ZROWS):
            pltpu.sync_copy(zbuf_v, acc.at[pl.ds(row_lo + m * ZROWS, ZROWS)])
        plsc.subcore_barrier()

        hq = h_hbm.at[cid * QPC + q]

        def fire_gather(c, buf, sem):
            for j in range(NSUB):
                pltpu.make_async_copy(
                    hq.at[src_v.at[c * NSUB + j]],
                    buf.at[pl.ds(j * SUB, SUB)], sem).start()

        def wait_gather(c, buf, sem):
            for j in range(NSUB):
                pltpu.make_async_copy(
                    hq.at[src_v.at[c * NSUB + j]],
                    buf.at[pl.ds(j * SUB, SUB)], sem).wait()

        def fire_scatter(c, buf, sem):
            for j in range(NSUB):
                pltpu.make_async_copy(
                    buf.at[pl.ds(j * SUB, SUB)],
                    acc.at[dst_v.at[c * NSUB + j]], sem).start(add=True)

        def wait_scatter(c, buf, sem):
            for j in range(NSUB):
                pltpu.make_async_copy(
                    buf.at[pl.ds(j * SUB, SUB)],
                    acc.at[dst_v.at[c * NSUB + j]], sem).wait()

        fire_gather(0, rows_a, gsem_a)

        def pair_body(t, _):
            c0 = 2 * t
            c1 = c0 + 1
            wait_gather(c0, rows_a, gsem_a)

            @pl.when(t > 0)
            def _():
                wait_scatter(c1 - 2, rows_b, ssem_b)
            fire_gather(c1, rows_b, gsem_b)
            scale(rows_a, c0)
            fire_scatter(c0, rows_a, ssem_a)
            wait_gather(c1, rows_b, gsem_b)

            @pl.when(t + 1 < CHUNKS // 2)
            def _():
                wait_scatter(c0, rows_a, ssem_a)
                fire_gather(c0 + 2, rows_a, gsem_a)
            scale(rows_b, c1)
            fire_scatter(c1, rows_b, ssem_b)
            return 0

        lax.fori_loop(0, CHUNKS // 2, pair_body, 0)
        wait_scatter(CHUNKS - 2, rows_a, ssem_a)
        wait_scatter(CHUNKS - 1, rows_b, ssem_b)
        plsc.subcore_barrier()

        col_lo = (cid * QPC + q) * DH
        for m in range(ROWS_PER_TILE // ZROWS):
            buf = rows_a if m % 2 == 0 else rows_b
            r_lo = row_lo + m * ZROWS
            pltpu.sync_copy(acc.at[pl.ds(r_lo, ZROWS)],
                            buf.at[pl.ds(0, ZROWS)])

            @plsc.parallel_loop(0, ZROWS, unroll=4)
            def leaky_body(r):
                for k in range(DH // 16):
                    v = buf[r, pl.ds(k * 16, 16)]
                    buf[r, pl.ds(k * 16, 16)] = jnp.where(v >= 0, v,
                                                          0.01 * v)
            pltpu.make_async_copy(
                buf.at[pl.ds(0, ZROWS)],
                out_hbm.at[pl.ds(r_lo, ZROWS), pl.ds(col_lo, DH)],
                gsem_a if m % 2 == 0 else gsem_b).start()
            if m >= 2:
                p_lo = row_lo + (m - 2) * ZROWS
                pltpu.make_async_copy(
                    buf.at[pl.ds(0, ZROWS)],
                    out_hbm.at[pl.ds(p_lo, ZROWS), pl.ds(col_lo, DH)],
                    gsem_a if m % 2 == 0 else gsem_b).wait()
        for m in (3, 4):
            buf = rows_a if m % 2 == 0 else rows_b
            r_lo = row_lo + m * ZROWS
            pltpu.make_async_copy(
                buf.at[pl.ds(0, ZROWS)],
                out_hbm.at[pl.ds(r_lo, ZROWS), pl.ds(col_lo, DH)],
                gsem_a if m % 2 == 0 else gsem_b).wait()
        if q + 1 < QPC:
            plsc.subcore_barrier()


def _sc_aggregate(h, edge3d, vals):
    mesh = plsc.VectorSubcoreMesh(core_axis_name="c", subcore_axis_name="s")
    return pl.kernel(
        _sc_body,
        out_type=jax.ShapeDtypeStruct((N, D), jnp.float32),
        mesh=mesh,
        compiler_params=pltpu.CompilerParams(use_tc_tiling_on_sc=False,
                                             needs_layout_passes=False),
        scratch_types=[
            pltpu.VMEM((IDX_ROWS, SUB), jnp.int32),
            pltpu.VMEM((IDX_ROWS, SUB), jnp.int32),
            pltpu.VMEM((EPT,), jnp.float32),
            pltpu.VMEM((C, DH), jnp.float32),
            pltpu.VMEM((C, DH), jnp.float32),
            pltpu.VMEM((ZROWS, DH), jnp.float32),
            pltpu.SemaphoreType.DMA,
            pltpu.SemaphoreType.DMA,
            pltpu.SemaphoreType.DMA,
            pltpu.SemaphoreType.DMA,
            pltpu.VMEM_SHARED((N, DH), jnp.float32),
        ],
    )(h, edge3d, vals)


def _mm_body(x_ref, w_ref, o_ref):
    h = jnp.dot(x_ref[...], w_ref[...], preferred_element_type=jnp.float32)
    for q in range(NQ):
        o_ref[q] = h[:, q * DH:(q + 1) * DH]


def _matmul(x, W):
    blk = 2000
    return pl.pallas_call(
        _mm_body,
        grid=(N // blk,),
        in_specs=[
            pl.BlockSpec((blk, D), lambda i: (i, 0)),
            pl.BlockSpec((D, D), lambda i: (0, 0)),
        ],
        out_specs=pl.BlockSpec((NQ, blk, DH), lambda i: (0, i, 0)),
        out_shape=jax.ShapeDtypeStruct((NQ, N, DH), jnp.float32),
    )(x, W)


def kernel(x, edge_index, adj_vals, W):
    h = _matmul(x, W)
    edge3d = edge_index.reshape(2, E // SUB, SUB)
    return _sc_aggregate(h, edge3d, adj_vals)
